# Initial kernel scaffold; baseline (speedup 1.0000x reference)
import jax
import jax.numpy as jnp
from jax.experimental import pallas as pl


def kernel(adj_indices, adj_values, features, W, bias, ln_gamma, ln_beta):
    raise NotImplementedError("write your pallas kernel here")


# speedup vs baseline: 4.4367x; 4.4367x over previous
import functools

import jax
import jax.numpy as jnp
from jax import lax
from jax.experimental import pallas as pl
from jax.experimental.pallas import tpu as pltpu
from jax.experimental.pallas import tpu_sc as plsc

_N = 10000
_E = 320000
_D = 128

_L = 16
_NC = 2
_NS = 16
_NW = _NC * _NS
_EPW = _E // _NW
_C = 80
_NP = 10240
_RPT = _NP // _NS
_ZR = 128


def _matmul_body(x_ref, w_ref, o_ref):
    o_ref[...] = jnp.dot(x_ref[...], w_ref[...],
                         preferred_element_type=jnp.float32)


def _matmul(x, w):
    bm = 1000
    return pl.pallas_call(
        _matmul_body,
        grid=(_N // bm,),
        in_specs=[
            pl.BlockSpec((bm, _D), lambda i: (i, 0)),
            pl.BlockSpec((_D, _D), lambda i: (0, 0)),
        ],
        out_specs=pl.BlockSpec((bm, _D), lambda i: (i, 0)),
        out_shape=jax.ShapeDtypeStruct((_N, _D), jnp.float32),
    )(x, w)


def _spmm_body(base_hbm, row_hbm, col_hbm, val_hbm, out_hbm,
               colv, rowv, valv, rowsv, zbuf, acc, sem):
    cid = lax.axis_index("c")
    sid = lax.axis_index("s")

    def _zrow(r, carry):
        for j in range(_D // _L):
            zbuf[r, pl.ds(j * _L, _L)] = jnp.zeros((_L,), jnp.float32)
        return carry

    lax.fori_loop(0, _ZR, _zrow, 0)
    for b in range(_RPT // _ZR):
        pltpu.sync_copy(zbuf, acc.at[pl.ds(sid * _RPT + b * _ZR, _ZR), :])
    plsc.subcore_barrier()

    ebase = (cid * _NS + sid) * _EPW

    def _chunk(t, carry):
        eb = ebase + t * _C
        pltpu.sync_copy(col_hbm.at[pl.ds(eb, _C)], colv)
        pltpu.sync_copy(row_hbm.at[pl.ds(eb, _C)], rowv)
        pltpu.sync_copy(val_hbm.at[pl.ds(eb, _C)], valv)
        pltpu.async_copy(base_hbm.at[colv], rowsv, sem).wait()

        def _scale(a, c2):
            va = valv[pl.ds(a * _L, _L)]
            for b in range(_L):
                vv = lax.gather(
                    va, jnp.full((_L, 1), b, jnp.int32),
                    lax.GatherDimensionNumbers(offset_dims=(),
                                               collapsed_slice_dims=(0,),
                                               start_index_map=(0,)),
                    (1,), mode=lax.GatherScatterMode.PROMISE_IN_BOUNDS)
                e = a * _L + b
                for j in range(_D // _L):
                    rowsv[e, pl.ds(j * _L, _L)] = (
                        rowsv[e, pl.ds(j * _L, _L)] * vv)
            return c2

        lax.fori_loop(0, _C // _L, _scale, 0)
        pltpu.sync_copy(rowsv, acc.at[rowv], add=True)
        return carry

    lax.fori_loop(0, _EPW // _C, _chunk, 0)
    plsc.subcore_barrier()

    pltpu.sync_copy(acc.at[pl.ds(sid * _RPT, _RPT), :],
                    out_hbm.at[cid, pl.ds(sid * _RPT, _RPT), :])


_spmm = functools.partial(
    pl.kernel,
    mesh=plsc.VectorSubcoreMesh(core_axis_name="c", subcore_axis_name="s"),
    out_type=jax.ShapeDtypeStruct((_NC, _NP, _D), jnp.float32),
    scratch_types=[
        pltpu.VMEM((_C,), jnp.int32),
        pltpu.VMEM((_C,), jnp.int32),
        pltpu.VMEM((_C,), jnp.float32),
        pltpu.VMEM((_C, _D), jnp.float32),
        pltpu.VMEM((_ZR, _D), jnp.float32),
        pltpu.VMEM_SHARED((_NP, _D), jnp.float32),
        pltpu.SemaphoreType.DMA,
    ],
)(_spmm_body)


def _epi_body(a0_ref, a1_ref, b_ref, g_ref, be_ref, o_ref):
    x = a0_ref[...] + a1_ref[...] + b_ref[...]
    act = jnp.where(x > 0, x, jnp.exp(jnp.minimum(x, 0.0)) - 1.0)
    mean = jnp.mean(act, axis=-1, keepdims=True)
    var = jnp.mean((act - mean) ** 2, axis=-1, keepdims=True)
    o_ref[...] = ((act - mean) * lax.rsqrt(var + 1e-5)) * g_ref[...] + be_ref[...]


def _epilogue(a0, a1, bias, gamma, beta):
    bm = 1000
    vec = pl.BlockSpec((1, _D), lambda i: (0, 0))
    blk = pl.BlockSpec((bm, _D), lambda i: (i, 0))
    return pl.pallas_call(
        _epi_body,
        grid=(_N // bm,),
        in_specs=[blk, blk, vec, vec, vec],
        out_specs=blk,
        out_shape=jax.ShapeDtypeStruct((_N, _D), jnp.float32),
    )(a0, a1, bias, gamma, beta)


def kernel(adj_indices, adj_values, features, W, bias, ln_gamma, ln_beta):
    row = adj_indices[0]
    col = adj_indices[1]
    base = _matmul(features, W)
    parts = _spmm(base, row, col, adj_values)
    return _epilogue(parts[0, :_N], parts[1, :_N], bias,
                     ln_gamma.reshape(1, _D), ln_beta.reshape(1, _D))
